# baseline (speedup 1.0000x reference)
import functools

import jax
import jax.numpy as jnp
from jax import lax
from jax.experimental import pallas as pl
from jax.experimental.pallas import tpu as pltpu
from jax.experimental.pallas import tpu_sc as plsc

B = 16384
D = 100
BLK = 2 * D
CHUNK = 128
LANES = 16


def _make_sc_call():
    info = plsc.get_sparse_core_info()
    nc, ns = info.num_cores, info.num_subcores
    nw = nc * ns
    b_per_w = B // nw
    n_chunks = b_per_w // CHUNK
    mesh = plsc.VectorSubcoreMesh(core_axis_name="c", subcore_axis_name="s")

    @functools.partial(
        pl.kernel,
        out_type=(
            jax.ShapeDtypeStruct((B,), jnp.float32),
            jax.ShapeDtypeStruct((B,), jnp.float32),
        ),
        mesh=mesh,
        compiler_params=pltpu.CompilerParams(use_tc_tiling_on_sc=False,
                                             needs_layout_passes=False),
        scratch_types=[
            pltpu.VMEM((CHUNK,), jnp.int32),
            pltpu.VMEM((CHUNK,), jnp.int32),
            pltpu.VMEM((CHUNK,), jnp.int32),
            pltpu.VMEM((CHUNK,), jnp.int32),
            pltpu.VMEM((CHUNK,), jnp.int32),
            pltpu.VMEM((CHUNK,), jnp.int32),
            pltpu.VMEM((CHUNK, BLK), jnp.float32),
            pltpu.VMEM((CHUNK, BLK), jnp.float32),
            pltpu.VMEM((CHUNK, BLK), jnp.float32),
            pltpu.VMEM((CHUNK,), jnp.float32),
            pltpu.VMEM((CHUNK,), jnp.float32),
            pltpu.SemaphoreType.DMA,
        ],
    )
    def sc_call(ub_hbm, pb_hbm, nb_hbm, uo_hbm, po_hbm, no_hbm,
                ut_hbm, it_hbm, pos_hbm, neg_hbm,
                idx_u, idx_p, idx_n, off_u, off_p, off_n,
                u_rows, p_rows, n_rows, pos_c, neg_c, sem):
        wid = lax.axis_index("s") * nc + lax.axis_index("c")
        lane = lax.iota(jnp.int32, LANES)
        zeros = jnp.zeros((LANES,), jnp.float32)

        for c in range(n_chunks):
            base = wid * b_per_w + c * CHUNK
            pltpu.sync_copy(ub_hbm.at[pl.ds(base, CHUNK)], idx_u)
            pltpu.sync_copy(pb_hbm.at[pl.ds(base, CHUNK)], idx_p)
            pltpu.sync_copy(nb_hbm.at[pl.ds(base, CHUNK)], idx_n)
            pltpu.sync_copy(uo_hbm.at[pl.ds(base, CHUNK)], off_u)
            pltpu.sync_copy(po_hbm.at[pl.ds(base, CHUNK)], off_p)
            pltpu.sync_copy(no_hbm.at[pl.ds(base, CHUNK)], off_n)
            cu = pltpu.async_copy(ut_hbm.at[idx_u], u_rows, sem)
            cp = pltpu.async_copy(it_hbm.at[idx_p], p_rows, sem)
            cn = pltpu.async_copy(it_hbm.at[idx_n], n_rows, sem)
            cu.wait()
            cp.wait()
            cn.wait()

            def group_body(g, _):
                rows = g * LANES + lane
                ov_u = off_u[pl.ds(g * LANES, LANES)]
                ov_p = off_p[pl.ds(g * LANES, LANES)]
                ov_n = off_n[pl.ds(g * LANES, LANES)]

                def d_step(d, carry):
                    acc_p, acc_n, cu_, cp_, cn_ = carry
                    u = plsc.load_gather(u_rows, [rows, cu_])
                    p = plsc.load_gather(p_rows, [rows, cp_])
                    n = plsc.load_gather(n_rows, [rows, cn_])
                    return (acc_p + u * p, acc_n + u * n,
                            cu_ + 1, cp_ + 1, cn_ + 1)

                acc_p, acc_n, _, _, _ = lax.fori_loop(
                    0, D, d_step, (zeros, zeros, ov_u, ov_p, ov_n),
                    unroll=4)
                pos_c[pl.ds(g * LANES, LANES)] = acc_p
                neg_c[pl.ds(g * LANES, LANES)] = acc_n
                return 0

            lax.fori_loop(0, CHUNK // LANES, group_body, 0)
            pltpu.sync_copy(pos_c, pos_hbm.at[pl.ds(base, CHUNK)])
            pltpu.sync_copy(neg_c, neg_hbm.at[pl.ds(base, CHUNK)])

    return sc_call


def kernel(user_inputs, pos_inputs, neg_inputs, user_table, item_table):
    ui = jnp.squeeze(user_inputs, axis=-1)
    pi = jnp.squeeze(pos_inputs, axis=-1)
    ni = jnp.squeeze(neg_inputs, axis=-1)
    nu = user_table.shape[0]
    nit = item_table.shape[0]
    ut2 = user_table.reshape(nu // 2, BLK)
    it2 = item_table.reshape(nit // 2, BLK)
    pos, neg = _make_sc_call()(
        ui >> 1, pi >> 1, ni >> 1,
        (ui & 1) * D, (pi & 1) * D, (ni & 1) * D,
        ut2, it2)
    return (pos[:, None], neg[:, None])


# speedup vs baseline: 2.8988x; 2.8988x over previous
import functools

import jax
import jax.numpy as jnp
from jax import lax
from jax.experimental import pallas as pl
from jax.experimental.pallas import tpu as pltpu
from jax.experimental.pallas import tpu_sc as plsc

B = 16384
D = 100
SUB = 8
G = 16
LANES = 16


def _make_sc_call():
    info = plsc.get_sparse_core_info()
    nc, ns = info.num_cores, info.num_subcores
    nw = nc * ns
    b_per_w = B // nw
    n_groups = b_per_w // G
    n_pairs = n_groups // 2
    mesh = plsc.VectorSubcoreMesh(core_axis_name="c", subcore_axis_name="s")

    @functools.partial(
        pl.kernel,
        out_type=(
            jax.ShapeDtypeStruct((B,), jnp.float32),
            jax.ShapeDtypeStruct((B,), jnp.float32),
        ),
        mesh=mesh,
        compiler_params=pltpu.CompilerParams(use_tc_tiling_on_sc=True,
                                             needs_layout_passes=False),
        scratch_types=[
            pltpu.VMEM((b_per_w,), jnp.int32),
            pltpu.VMEM((b_per_w,), jnp.int32),
            pltpu.VMEM((b_per_w,), jnp.int32),
            pltpu.VMEM((2, G, SUB, D), jnp.float32),
            pltpu.VMEM((2, G, SUB, D), jnp.float32),
            pltpu.VMEM((2, G, SUB, D), jnp.float32),
            pltpu.VMEM((b_per_w,), jnp.float32),
            pltpu.VMEM((b_per_w,), jnp.float32),
            pltpu.SemaphoreType.DMA,
            pltpu.SemaphoreType.DMA,
            pltpu.SemaphoreType.DMA,
            pltpu.SemaphoreType.DMA,
            pltpu.SemaphoreType.DMA,
            pltpu.SemaphoreType.DMA,
        ],
    )
    def sc_call(ui_hbm, pi_hbm, ni_hbm, ut_hbm, it_hbm, pos_hbm, neg_hbm,
                idx_u, idx_p, idx_n, grp_u, grp_p, grp_n, pos_c, neg_c,
                su0, su1, sp0, sp1, sn0, sn1):
        wid = lax.axis_index("s") * nc + lax.axis_index("c")
        base = wid * b_per_w
        lane = lax.iota(jnp.int32, LANES)
        zeros = jnp.zeros((LANES,), jnp.float32)
        tail_mask = jnp.where(lane >= LANES - (D % LANES), 1.0, 0.0)
        bfly = [lane ^ s for s in (8, 4, 2, 1)]
        sems = ((su0, su1), (sp0, sp1), (sn0, sn1))

        def vsum(x):
            for idx in bfly:
                x = x + x.at[idx].get(mode="promise_in_bounds")
            return x

        pltpu.sync_copy(ui_hbm.at[pl.ds(base, b_per_w)], idx_u)
        pltpu.sync_copy(pi_hbm.at[pl.ds(base, b_per_w)], idx_p)
        pltpu.sync_copy(ni_hbm.at[pl.ds(base, b_per_w)], idx_n)

        def load_group_idx(g):
            return (idx_u[pl.ds(g * G, G)],
                    idx_p[pl.ds(g * G, G)],
                    idx_n[pl.ds(g * G, G)])

        def issue(g, buf):
            vgu, vgp, vgn = load_group_idx(g)
            for j in range(G):
                pltpu.async_copy(ut_hbm.at[vgu[j] >> 3], grp_u.at[buf, j],
                                 sems[0][buf])
                pltpu.async_copy(it_hbm.at[vgp[j] >> 3], grp_p.at[buf, j],
                                 sems[1][buf])
                pltpu.async_copy(it_hbm.at[vgn[j] >> 3], grp_n.at[buf, j],
                                 sems[2][buf])

        def drain(buf):
            for j in range(G):
                pltpu.make_async_copy(ut_hbm.at[0], grp_u.at[buf, j],
                                      sems[0][buf]).wait()
                pltpu.make_async_copy(it_hbm.at[0], grp_p.at[buf, j],
                                      sems[1][buf]).wait()
                pltpu.make_async_copy(it_hbm.at[0], grp_n.at[buf, j],
                                      sems[2][buf]).wait()

        def compute(g, buf):
            vgu, vgp, vgn = load_group_idx(g)
            res_p = zeros
            res_n = zeros
            for j in range(G):
                s_u = vgu[j] & 7
                s_p = vgp[j] & 7
                s_n = vgn[j] & 7
                acc_p = zeros
                acc_n = zeros
                for k in range(D // LANES):
                    u = grp_u[buf, j, s_u, pl.ds(k * LANES, LANES)]
                    p = grp_p[buf, j, s_p, pl.ds(k * LANES, LANES)]
                    n = grp_n[buf, j, s_n, pl.ds(k * LANES, LANES)]
                    acc_p = acc_p + u * p
                    acc_n = acc_n + u * n
                u = grp_u[buf, j, s_u, pl.ds(D - LANES, LANES)] * tail_mask
                p = grp_p[buf, j, s_p, pl.ds(D - LANES, LANES)]
                n = grp_n[buf, j, s_n, pl.ds(D - LANES, LANES)]
                acc_p = acc_p + u * p
                acc_n = acc_n + u * n
                m = lane == j
                res_p = jnp.where(m, vsum(acc_p), res_p)
                res_n = jnp.where(m, vsum(acc_n), res_n)
            pos_c[pl.ds(g * G, G)] = res_p
            neg_c[pl.ds(g * G, G)] = res_n

        issue(0, 0)
        issue(1, 1)

        def pair_body(p, _):
            for buf in range(2):
                g = 2 * p + buf
                drain(buf)
                compute(g, buf)

                @pl.when(p < n_pairs - 1)
                def _():
                    issue(g + 2, buf)

            return 0

        lax.fori_loop(0, n_pairs, pair_body, 0)
        pltpu.sync_copy(pos_c, pos_hbm.at[pl.ds(base, b_per_w)])
        pltpu.sync_copy(neg_c, neg_hbm.at[pl.ds(base, b_per_w)])

    return sc_call


def kernel(user_inputs, pos_inputs, neg_inputs, user_table, item_table):
    ui = jnp.squeeze(user_inputs, axis=-1)
    pi = jnp.squeeze(pos_inputs, axis=-1)
    ni = jnp.squeeze(neg_inputs, axis=-1)
    ut3 = user_table.reshape(user_table.shape[0] // SUB, SUB, D)
    it3 = item_table.reshape(item_table.shape[0] // SUB, SUB, D)
    pos, neg = _make_sc_call()(ui, pi, ni, ut3, it3)
    return (pos[:, None], neg[:, None])
